# Initial kernel scaffold; baseline (speedup 1.0000x reference)
import functools

import jax
import jax.numpy as jnp
from jax import lax
from jax.experimental import pallas as pl
from jax.experimental.pallas import tpu as pltpu
from jax.experimental.pallas import tpu_sc as plsc

B = 16384
F = 26
V = 1000000
D = 13

NC = 2
NS = 16
NW = NC * NS
EPW = B // NW
LANE = 16
IDXW = 128
ROWS = EPW * F // IDXW
RPF = EPW // IDXW

_mesh = plsc.VectorSubcoreMesh(
    core_axis_name="c", subcore_axis_name="s", num_cores=NC, num_subcores=NS
)


@functools.partial(
    pl.kernel,
    out_type=jax.ShapeDtypeStruct((B,), jnp.float32),
    mesh=_mesh,
    scratch_types=[
        pltpu.VMEM((ROWS, IDXW), jnp.int32),
        pltpu.VMEM((ROWS, IDXW), jnp.float32),
        pltpu.VMEM((D, EPW), jnp.float32),
        pltpu.VMEM((EPW,), jnp.float32),
        pltpu.VMEM((LANE,), jnp.float32),
        pltpu.VMEM((LANE,), jnp.float32),
        pltpu.SemaphoreType.DMA,
    ],
)
def _sc_call(ids_hbm, dense_hbm, flat_hbm, wd_hbm, bias_hbm, out_hbm,
             idx_v, vals_v, dvals_v, out_v, wd_v, bias_v, sem):
    wid = lax.axis_index("s") * NC + lax.axis_index("c")

    pltpu.sync_copy(ids_hbm.at[wid], idx_v)
    pltpu.sync_copy(dense_hbm.at[wid], dvals_v)
    pltpu.sync_copy(wd_hbm, wd_v)
    pltpu.sync_copy(bias_hbm, bias_v)

    def _off_row(r, _):
        off = (r // RPF) * V

        def _off_chunk(k, _):
            sl = pl.ds(k * LANE, LANE)
            idx_v[r, sl] = idx_v[r, sl] + off
            return 0

        return lax.fori_loop(0, IDXW // LANE, _off_chunk, 0)

    lax.fori_loop(0, ROWS, _off_row, 0)

    pltpu.async_copy(flat_hbm.at[idx_v], vals_v, sem).wait()

    wsplat = [
        plsc.load_gather(wd_v, [jnp.full((LANE,), d, jnp.int32)])
        for d in range(D)
    ]
    bias_splat = plsc.load_gather(bias_v, [jnp.zeros((LANE,), jnp.int32)])

    def _chunk(t, _):
        q = t // (IDXW // LANE)
        k = t % (IDXW // LANE)
        row_sl = pl.ds(k * LANE, LANE)
        ex_sl = pl.ds(q * IDXW + k * LANE, LANE)

        def _fsum(f, acc):
            return acc + vals_v[f * RPF + q, row_sl]

        acc = lax.fori_loop(0, F, _fsum, jnp.zeros((LANE,), jnp.float32))
        for d in range(D):
            acc = acc + dvals_v[d, ex_sl] * wsplat[d]
        acc = acc + bias_splat
        out_v[ex_sl] = 1.0 / (1.0 + jnp.exp(-acc))
        return 0

    lax.fori_loop(0, EPW // LANE, _chunk, 0)

    pltpu.sync_copy(out_v, out_hbm.at[pl.ds(wid * EPW, EPW)])


def kernel(sparse_ids, dense_features, W_cat, W_dense, bias):
    ids = sparse_ids.astype(jnp.int32)
    ids = ids.reshape(NW, EPW, F).transpose(0, 2, 1).reshape(NW, ROWS, IDXW)
    dense_t = dense_features.reshape(NW, EPW, D).transpose(0, 2, 1)
    flat_w = W_cat.reshape(-1)
    wd_pad = jnp.pad(W_dense.reshape(-1), (0, LANE - D))
    bias_pad = jnp.pad(bias.reshape(-1), (0, LANE - 1))
    out = _sc_call(ids, dense_t, flat_w, wd_pad, bias_pad)
    return out.reshape(B, 1)


# speedup vs baseline: 1.0180x; 1.0180x over previous
import functools

import jax
import jax.numpy as jnp
from jax import lax
from jax.experimental import pallas as pl
from jax.experimental.pallas import tpu as pltpu
from jax.experimental.pallas import tpu_sc as plsc

B = 16384
F = 26
V = 1000000
D = 13

NC = 2
NS = 16
NW = NC * NS
EPW = B // NW
LANE = 16
IPW = EPW * F

_mesh = plsc.VectorSubcoreMesh(
    core_axis_name="c", subcore_axis_name="s", num_cores=NC, num_subcores=NS
)


@functools.partial(
    pl.kernel,
    out_type=jax.ShapeDtypeStruct((B,), jnp.float32),
    mesh=_mesh,
    scratch_types=[
        pltpu.VMEM((IPW,), jnp.int32),
        pltpu.VMEM((IPW,), jnp.float32),
        pltpu.VMEM((D, EPW), jnp.float32),
        pltpu.VMEM((EPW,), jnp.float32),
        pltpu.VMEM((D, LANE), jnp.float32),
        pltpu.VMEM((LANE,), jnp.float32),
        pltpu.SemaphoreType.DMA,
    ],
)
def _sc_call(ids_hbm, dense_hbm, flat_hbm, wd_hbm, bias_hbm, out_hbm,
             idx_v, vals_v, dvals_v, out_v, wd_v, bias_v, sem):
    wid = lax.axis_index("s") * NC + lax.axis_index("c")

    pltpu.sync_copy(ids_hbm.at[wid], idx_v)
    pltpu.sync_copy(dense_hbm.at[wid], dvals_v)
    pltpu.sync_copy(wd_hbm, wd_v)
    pltpu.sync_copy(bias_hbm, bias_v)

    def _off_chunk(i, _):
        off = (i // (EPW // LANE)) * V
        sl = pl.ds(i * LANE, LANE)
        idx_v[sl] = idx_v[sl] + off
        return 0

    lax.fori_loop(0, IPW // LANE, _off_chunk, 0)

    pltpu.async_copy(flat_hbm.at[idx_v], vals_v, sem).wait()

    wsplat = [wd_v[d, :] for d in range(D)]
    bias_splat = bias_v[...]

    def _chunk(t, _):
        ex_sl = pl.ds(t * LANE, LANE)

        def _fsum(f, acc):
            return acc + vals_v[pl.ds(f * EPW + t * LANE, LANE)]

        acc = lax.fori_loop(0, F, _fsum, jnp.zeros((LANE,), jnp.float32))
        for d in range(D):
            acc = acc + dvals_v[d, ex_sl] * wsplat[d]
        acc = acc + bias_splat
        out_v[ex_sl] = 1.0 / (1.0 + jnp.exp(-acc))
        return 0

    lax.fori_loop(0, EPW // LANE, _chunk, 0)

    pltpu.sync_copy(out_v, out_hbm.at[pl.ds(wid * EPW, EPW)])


def kernel(sparse_ids, dense_features, W_cat, W_dense, bias):
    ids = sparse_ids.astype(jnp.int32)
    ids = ids.reshape(NW, EPW, F).transpose(0, 2, 1).reshape(NW, IPW)
    dense_t = dense_features.reshape(NW, EPW, D).transpose(0, 2, 1)
    flat_w = W_cat.reshape(-1)
    wd_bc = jnp.broadcast_to(W_dense.reshape(D, 1), (D, LANE))
    bias_bc = jnp.broadcast_to(bias.reshape(1), (LANE,))
    out = _sc_call(ids, dense_t, flat_w, wd_bc, bias_bc)
    return out.reshape(B, 1)
